# Initial kernel scaffold; baseline (speedup 1.0000x reference)
import jax
import jax.numpy as jnp
from jax.experimental import pallas as pl


def kernel(x, W_gate0, b_gate0, W_cand0, b_cand0, W_gate1, b_gate1, W_cand1, b_cand1, W_fc, b_fc):
    raise NotImplementedError("write your pallas kernel here")


# speedup vs baseline: 2.1807x; 2.1807x over previous
import jax
import jax.numpy as jnp
from jax.experimental import pallas as pl
from jax.experimental.pallas import tpu as pltpu

_B, _T, _N, _D, _H = 16, 12, 128, 64, 128
_NUM_MAT = 5


def _mm(a, b):
    return jax.lax.dot_general(a, b, (((1,), (0,)), ((), ())),
                               preferred_element_type=jnp.float32)


def _mmt(a, b):
    return jax.lax.dot_general(a, b, (((1,), (1,)), ((), ())),
                               preferred_element_type=jnp.float32)


def _body(x_ref, wg0i_ref, wg0s_ref, bg0_ref, wc0i_ref, wc0s_ref, bc0_ref,
          wg1i_ref, wg1s_ref, bg1_ref, wc1i_ref, wc1s_ref, bc1_ref,
          wfc_ref, bfc_ref, out_ref):
    xc = x_ref[0]

    mu = jnp.sum(jnp.sum(xc, axis=0), axis=1, keepdims=True) * (1.0 / (_T * _D))
    xm = xc - mu[None]
    cov = _mmt(xm[0], xm[0])
    for t in range(1, _T):
        cov = cov + _mmt(xm[t], xm[t])
    sq = jnp.sum(jnp.sum(xm * xm, axis=0), axis=1, keepdims=True)
    var = jnp.sqrt(jnp.maximum(sq, 1e-12))
    adj = jnp.abs(cov) / (var * jnp.transpose(var))
    s0 = adj / jnp.sum(adj, axis=1, keepdims=True)
    adjt = jnp.transpose(adj)
    s1 = adjt / jnp.transpose(jnp.sum(adj, axis=0, keepdims=True))

    a20 = _mm(s0, s0) * 2.0
    a21 = _mm(s1, s1) * 2.0

    def diffuse(v):
        return (_mm(s0, v), _mm(a20, v) - v, _mm(s1, v), _mm(a21, v) - v)

    wg0i = wg0i_ref[...]
    wg0s = wg0s_ref[...]
    bg0 = bg0_ref[...]
    wc0i = wc0i_ref[...]
    wc0s = wc0s_ref[...]
    bc0 = bc0_ref[...]
    wg1i = wg1i_ref[...]
    wg1s = wg1s_ref[...]
    bg1 = bg1_ref[...]
    wc1i = wc1i_ref[...]
    wc1s = wc1s_ref[...]
    bc1 = bc1_ref[...]

    def gru(ipcat, h, wgi, wgs, bg, wci, wcs, bc):
        stcat = jnp.concatenate((h,) + diffuse(h), axis=1)
        g = jax.nn.sigmoid(_mm(ipcat, wgi) + _mm(stcat, wgs) + bg)
        r = g[:, :_H]
        u = g[:, _H:]
        rh = r * h
        rcat = jnp.concatenate((rh,) + diffuse(rh), axis=1)
        c = jnp.tanh(_mm(ipcat, wci) + _mm(rcat, wcs) + bc)
        return u * h + (1.0 - u) * c

    h0 = jnp.zeros((_N, _H), jnp.float32)
    h1 = jnp.zeros((_N, _H), jnp.float32)
    for t in range(_T):
        xt = xc[t]
        ipcat0 = jnp.concatenate((xt,) + diffuse(xt), axis=1)
        h0 = gru(ipcat0, h0, wg0i, wg0s, bg0, wc0i, wc0s, bc0)
        ipcat1 = jnp.concatenate((h0,) + diffuse(h0), axis=1)
        h1 = gru(ipcat1, h1, wg1i, wg1s, bg1, wc1i, wc1s, bc1)

    lg = jnp.sum(jnp.maximum(h1, 0.0) * wfc_ref[...], axis=1,
                 keepdims=True) + bfc_ref[...]
    out_ref[...] = jnp.full((1, 1, _N), jnp.max(lg), jnp.float32)


def _split_w(w, din):
    wr = w.reshape(_NUM_MAT, din + _H, -1)
    w_in = wr[:, :din, :].reshape(_NUM_MAT * din, -1)
    w_st = wr[:, din:, :].reshape(_NUM_MAT * _H, -1)
    return w_in, w_st


@jax.jit
def kernel(x, W_gate0, b_gate0, W_cand0, b_cand0, W_gate1, b_gate1,
           W_cand1, b_cand1, W_fc, b_fc):
    wg0i, wg0s = _split_w(W_gate0, _D)
    wc0i, wc0s = _split_w(W_cand0, _D)
    wg1i, wg1s = _split_w(W_gate1, _H)
    wc1i, wc1s = _split_w(W_cand1, _H)

    const = lambda b: (0, 0)
    wspec = lambda a: pl.BlockSpec(a.shape, const)
    operands = (x, wg0i, wg0s, b_gate0.reshape(1, -1),
                wc0i, wc0s, b_cand0.reshape(1, -1),
                wg1i, wg1s, b_gate1.reshape(1, -1),
                wc1i, wc1s, b_cand1.reshape(1, -1),
                W_fc.reshape(1, _H), b_fc.reshape(1, 1))
    in_specs = [pl.BlockSpec((1, _T, _N, _D), lambda b: (b, 0, 0, 0))]
    in_specs += [wspec(a) for a in operands[1:]]

    out = pl.pallas_call(
        _body,
        grid=(_B,),
        in_specs=in_specs,
        out_specs=pl.BlockSpec((1, 1, _N), lambda b: (b, 0, 0)),
        out_shape=jax.ShapeDtypeStruct((_B, 1, _N), jnp.float32),
        compiler_params=pltpu.CompilerParams(
            dimension_semantics=("parallel",)),
    )(*operands)
    return out[:, 0, 0]


# speedup vs baseline: 2.9824x; 1.3677x over previous
import jax
import jax.numpy as jnp
from jax.experimental import pallas as pl
from jax.experimental.pallas import tpu as pltpu

_B, _T, _N, _D, _H = 16, 12, 128, 64, 128
_NUM_MAT = 5
_SPB = 4


def _mm(a, b):
    return jax.lax.dot_general(a, b, (((1,), (0,)), ((), ())),
                               preferred_element_type=jnp.float32)


def _mmt(a, b):
    return jax.lax.dot_general(a, b, (((1,), (1,)), ((), ())),
                               preferred_element_type=jnp.float32)


def _body(x_ref, wg0i_ref, wg0s_ref, bg0_ref, wc0i_ref, wc0s_ref, bc0_ref,
          wg1i_ref, wg1s_ref, bg1_ref, wc1i_ref, wc1s_ref, bc1_ref,
          wfc_ref, bfc_ref, out_ref):
    sup = []
    for i in range(_SPB):
        xc = x_ref[i]
        mu = jnp.sum(jnp.sum(xc, axis=0), axis=1, keepdims=True) \
            * (1.0 / (_T * _D))
        xm = xc - mu[None]
        cov = _mmt(xm[0], xm[0])
        for t in range(1, _T):
            cov = cov + _mmt(xm[t], xm[t])
        sq = jnp.sum(jnp.sum(xm * xm, axis=0), axis=1, keepdims=True)
        var = jnp.sqrt(jnp.maximum(sq, 1e-12))
        adj = jnp.abs(cov) / (var * jnp.transpose(var))
        s0 = adj / jnp.sum(adj, axis=1, keepdims=True)
        adjt = jnp.transpose(adj)
        s1 = adjt / jnp.transpose(jnp.sum(adj, axis=0, keepdims=True))
        sup.append((s0, _mm(s0, s0) * 2.0, s1, _mm(s1, s1) * 2.0))

    def diffuse(v):
        outs = []
        for i in range(_SPB):
            s0, a20, s1, a21 = sup[i]
            vi = v[i * _N:(i + 1) * _N]
            outs.append((_mm(s0, vi), _mm(a20, vi) - vi,
                         _mm(s1, vi), _mm(a21, vi) - vi))
        return tuple(
            jnp.concatenate([outs[i][k] for i in range(_SPB)], axis=0)
            for k in range(4))

    wg0i = wg0i_ref[...]
    wg0s = wg0s_ref[...]
    bg0 = bg0_ref[...]
    wc0i = wc0i_ref[...]
    wc0s = wc0s_ref[...]
    bc0 = bc0_ref[...]
    wg1i = wg1i_ref[...]
    wg1s = wg1s_ref[...]
    bg1 = bg1_ref[...]
    wc1i = wc1i_ref[...]
    wc1s = wc1s_ref[...]
    bc1 = bc1_ref[...]

    def gru(ipcat, h, wgi, wgs, bg, wci, wcs, bc):
        stcat = jnp.concatenate((h,) + diffuse(h), axis=1)
        g = jax.nn.sigmoid(_mm(ipcat, wgi) + _mm(stcat, wgs) + bg)
        r = g[:, :_H]
        u = g[:, _H:]
        rh = r * h
        rcat = jnp.concatenate((rh,) + diffuse(rh), axis=1)
        c = jnp.tanh(_mm(ipcat, wci) + _mm(rcat, wcs) + bc)
        return u * h + (1.0 - u) * c

    h0 = jnp.zeros((_SPB * _N, _H), jnp.float32)
    h1 = jnp.zeros((_SPB * _N, _H), jnp.float32)
    for t in range(_T):
        xt = jnp.concatenate([x_ref[i, t] for i in range(_SPB)], axis=0)
        ipcat0 = jnp.concatenate((xt,) + diffuse(xt), axis=1)
        h0 = gru(ipcat0, h0, wg0i, wg0s, bg0, wc0i, wc0s, bc0)
        ipcat1 = jnp.concatenate((h0,) + diffuse(h0), axis=1)
        h1 = gru(ipcat1, h1, wg1i, wg1s, bg1, wc1i, wc1s, bc1)

    lg = jnp.sum(jnp.maximum(h1, 0.0) * wfc_ref[...], axis=1,
                 keepdims=True) + bfc_ref[...]
    out_ref[...] = jnp.concatenate(
        [jnp.full((1, 1, _N), jnp.max(lg[i * _N:(i + 1) * _N]), jnp.float32)
         for i in range(_SPB)], axis=0)


def _split_w(w, din):
    wr = w.reshape(_NUM_MAT, din + _H, -1)
    w_in = wr[:, :din, :].reshape(_NUM_MAT * din, -1)
    w_st = wr[:, din:, :].reshape(_NUM_MAT * _H, -1)
    return w_in, w_st


@jax.jit
def kernel(x, W_gate0, b_gate0, W_cand0, b_cand0, W_gate1, b_gate1,
           W_cand1, b_cand1, W_fc, b_fc):
    wg0i, wg0s = _split_w(W_gate0, _D)
    wc0i, wc0s = _split_w(W_cand0, _D)
    wg1i, wg1s = _split_w(W_gate1, _H)
    wc1i, wc1s = _split_w(W_cand1, _H)

    const = lambda b: (0, 0)
    wspec = lambda a: pl.BlockSpec(a.shape, const)
    operands = (x, wg0i, wg0s, b_gate0.reshape(1, -1),
                wc0i, wc0s, b_cand0.reshape(1, -1),
                wg1i, wg1s, b_gate1.reshape(1, -1),
                wc1i, wc1s, b_cand1.reshape(1, -1),
                W_fc.reshape(1, _H), b_fc.reshape(1, 1))
    in_specs = [pl.BlockSpec((_SPB, _T, _N, _D), lambda b: (b, 0, 0, 0))]
    in_specs += [wspec(a) for a in operands[1:]]

    out = pl.pallas_call(
        _body,
        grid=(_B // _SPB,),
        in_specs=in_specs,
        out_specs=pl.BlockSpec((_SPB, 1, _N), lambda b: (b, 0, 0)),
        out_shape=jax.ShapeDtypeStruct((_B, 1, _N), jnp.float32),
        compiler_params=pltpu.CompilerParams(
            dimension_semantics=("parallel",)),
    )(*operands)
    return out[:, 0, 0]


# speedup vs baseline: 3.4582x; 1.1595x over previous
import jax
import jax.numpy as jnp
from jax.experimental import pallas as pl
from jax.experimental.pallas import tpu as pltpu

_B, _T, _N, _D, _H = 16, 12, 128, 64, 128
_NUM_MAT = 5
_SPB = 4


def _mm(a, b):
    return jax.lax.dot_general(a, b, (((1,), (0,)), ((), ())),
                               preferred_element_type=jnp.float32)


def _mmt(a, b):
    return jax.lax.dot_general(a, b, (((1,), (1,)), ((), ())),
                               preferred_element_type=jnp.float32)


def _body(x_ref, wg0i_ref, wg0s_ref, bg0_ref, wc0i_ref, wc0s_ref, bc0_ref,
          wg1i_ref, wg1s_ref, bg1_ref, wc1i_ref, wc1s_ref, bc1_ref,
          wfc_ref, bfc_ref, out_ref):
    row = jax.lax.broadcasted_iota(jnp.int32, (_N, _N), 0)
    col = jax.lax.broadcasted_iota(jnp.int32, (_N, _N), 1)
    eye = jnp.where(row == col, 1.0, 0.0).astype(jnp.float32)
    sup = []
    for i in range(_SPB):
        xc = x_ref[i]
        mu = jnp.sum(jnp.sum(xc, axis=0), axis=1, keepdims=True) \
            * (1.0 / (_T * _D))
        xm = xc - mu[None]
        cov = _mmt(xm[0], xm[0])
        for t in range(1, _T):
            cov = cov + _mmt(xm[t], xm[t])
        sq = jnp.sum(jnp.sum(xm * xm, axis=0), axis=1, keepdims=True)
        var = jnp.sqrt(jnp.maximum(sq, 1e-12))
        adj = jnp.abs(cov) / (var * jnp.transpose(var))
        s0 = adj / jnp.sum(adj, axis=1, keepdims=True)
        adjt = jnp.transpose(adj)
        s1 = adjt / jnp.transpose(jnp.sum(adj, axis=0, keepdims=True))
        sup.append(jnp.concatenate(
            [s0, _mm(s0, s0) * 2.0 - eye, s1, _mm(s1, s1) * 2.0 - eye],
            axis=0))

    def diffuse(v):
        prods = [_mm(sup[i], v[i * _N:(i + 1) * _N]) for i in range(_SPB)]
        return tuple(
            jnp.concatenate([p[k * _N:(k + 1) * _N] for p in prods], axis=0)
            for k in range(4))

    wg0i = wg0i_ref[...]
    wg0s = wg0s_ref[...]
    bg0 = bg0_ref[...]
    wc0i = wc0i_ref[...]
    wc0s = wc0s_ref[...]
    bc0 = bc0_ref[...]
    wg1i = wg1i_ref[...]
    wg1s = wg1s_ref[...]
    bg1 = bg1_ref[...]
    wc1i = wc1i_ref[...]
    wc1s = wc1s_ref[...]
    bc1 = bc1_ref[...]

    def gru(ipcat, h, wgi, wgs, bg, wci, wcs, bc):
        stcat = jnp.concatenate((h,) + diffuse(h), axis=1)
        g = jax.nn.sigmoid(_mm(ipcat, wgi) + _mm(stcat, wgs) + bg)
        r = g[:, :_H]
        u = g[:, _H:]
        rh = r * h
        rcat = jnp.concatenate((rh,) + diffuse(rh), axis=1)
        c = jnp.tanh(_mm(ipcat, wci) + _mm(rcat, wcs) + bc)
        return u * h + (1.0 - u) * c

    h0 = jnp.zeros((_SPB * _N, _H), jnp.float32)
    h1 = jnp.zeros((_SPB * _N, _H), jnp.float32)
    for t in range(_T):
        xt = jnp.concatenate([x_ref[i, t] for i in range(_SPB)], axis=0)
        ipcat0 = jnp.concatenate((xt,) + diffuse(xt), axis=1)
        h0 = gru(ipcat0, h0, wg0i, wg0s, bg0, wc0i, wc0s, bc0)
        ipcat1 = jnp.concatenate((h0,) + diffuse(h0), axis=1)
        h1 = gru(ipcat1, h1, wg1i, wg1s, bg1, wc1i, wc1s, bc1)

    lg = jnp.sum(jnp.maximum(h1, 0.0) * wfc_ref[...], axis=1,
                 keepdims=True) + bfc_ref[...]
    out_ref[...] = jnp.concatenate(
        [jnp.full((1, 1, _N), jnp.max(lg[i * _N:(i + 1) * _N]), jnp.float32)
         for i in range(_SPB)], axis=0)


def _split_w(w, din):
    wr = w.reshape(_NUM_MAT, din + _H, -1)
    w_in = wr[:, :din, :].reshape(_NUM_MAT * din, -1)
    w_st = wr[:, din:, :].reshape(_NUM_MAT * _H, -1)
    return w_in, w_st


@jax.jit
def kernel(x, W_gate0, b_gate0, W_cand0, b_cand0, W_gate1, b_gate1,
           W_cand1, b_cand1, W_fc, b_fc):
    wg0i, wg0s = _split_w(W_gate0, _D)
    wc0i, wc0s = _split_w(W_cand0, _D)
    wg1i, wg1s = _split_w(W_gate1, _H)
    wc1i, wc1s = _split_w(W_cand1, _H)

    const = lambda b: (0, 0)
    wspec = lambda a: pl.BlockSpec(a.shape, const)
    operands = (x, wg0i, wg0s, b_gate0.reshape(1, -1),
                wc0i, wc0s, b_cand0.reshape(1, -1),
                wg1i, wg1s, b_gate1.reshape(1, -1),
                wc1i, wc1s, b_cand1.reshape(1, -1),
                W_fc.reshape(1, _H), b_fc.reshape(1, 1))
    in_specs = [pl.BlockSpec((_SPB, _T, _N, _D), lambda b: (b, 0, 0, 0))]
    in_specs += [wspec(a) for a in operands[1:]]

    out = pl.pallas_call(
        _body,
        grid=(_B // _SPB,),
        in_specs=in_specs,
        out_specs=pl.BlockSpec((_SPB, 1, _N), lambda b: (b, 0, 0)),
        out_shape=jax.ShapeDtypeStruct((_B, 1, _N), jnp.float32),
        compiler_params=pltpu.CompilerParams(
            dimension_semantics=("parallel",)),
    )(*operands)
    return out[:, 0, 0]


# speedup vs baseline: 3.6634x; 1.0593x over previous
import jax
import jax.numpy as jnp
from jax.experimental import pallas as pl
from jax.experimental.pallas import tpu as pltpu

_B, _T, _N, _D, _H = 16, 12, 128, 64, 128
_NUM_MAT = 5
_SPB = 8


def _mm(a, b):
    return jax.lax.dot_general(a, b, (((1,), (0,)), ((), ())),
                               preferred_element_type=jnp.float32)


def _mmt(a, b):
    return jax.lax.dot_general(a, b, (((1,), (1,)), ((), ())),
                               preferred_element_type=jnp.float32)


def _body(x_ref, wi0_ref, wg0s_ref, bg0_ref, wc0s_ref, bc0_ref,
          wi1_ref, wg1s_ref, bg1_ref, wc1s_ref, bc1_ref,
          wfc_ref, bfc_ref, out_ref):
    row = jax.lax.broadcasted_iota(jnp.int32, (_N, _N), 0)
    col = jax.lax.broadcasted_iota(jnp.int32, (_N, _N), 1)
    eye = jnp.where(row == col, 1.0, 0.0).astype(jnp.float32)
    sup = []
    for i in range(_SPB):
        xc = x_ref[i]
        mu = jnp.sum(jnp.sum(xc, axis=0), axis=1, keepdims=True) \
            * (1.0 / (_T * _D))
        xm = xc - mu[None]
        cov = _mmt(xm[0], xm[0])
        for t in range(1, _T):
            cov = cov + _mmt(xm[t], xm[t])
        sq = jnp.sum(jnp.sum(xm * xm, axis=0), axis=1, keepdims=True)
        var = jnp.sqrt(jnp.maximum(sq, 1e-12))
        adj = jnp.abs(cov) / (var * jnp.transpose(var))
        s0 = adj / jnp.sum(adj, axis=1, keepdims=True)
        adjt = jnp.transpose(adj)
        s1 = adjt / jnp.transpose(jnp.sum(adj, axis=0, keepdims=True))
        sup.append(jnp.concatenate(
            [s0, _mm(s0, s0) * 2.0 - eye, s1, _mm(s1, s1) * 2.0 - eye],
            axis=0))

    def diffuse(v):
        prods = [_mm(sup[i], v[i * _N:(i + 1) * _N]) for i in range(_SPB)]
        return (v,) + tuple(
            jnp.concatenate([p[k * _N:(k + 1) * _N] for p in prods], axis=0)
            for k in range(4))

    wi0 = wi0_ref[...]
    wg0s = wg0s_ref[...]
    bg0 = bg0_ref[...]
    wc0s = wc0s_ref[...]
    bc0 = bc0_ref[...]
    wi1 = wi1_ref[...]
    wg1s = wg1s_ref[...]
    bg1 = bg1_ref[...]
    wc1s = wc1s_ref[...]
    bc1 = bc1_ref[...]

    def gru(ipcat, h, wi, wgs, bg, wcs, bc):
        ipg = _mm(ipcat, wi)
        stcat = jnp.concatenate(diffuse(h), axis=1)
        g = jax.nn.sigmoid(ipg[:, :2 * _H] + _mm(stcat, wgs) + bg)
        r = g[:, :_H]
        u = g[:, _H:]
        rh = r * h
        rcat = jnp.concatenate(diffuse(rh), axis=1)
        c = jnp.tanh(ipg[:, 2 * _H:] + _mm(rcat, wcs) + bc)
        return u * h + (1.0 - u) * c

    h0 = jnp.zeros((_SPB * _N, _H), jnp.float32)
    h1 = jnp.zeros((_SPB * _N, _H), jnp.float32)
    for t in range(_T):
        xt = jnp.concatenate([x_ref[i, t] for i in range(_SPB)], axis=0)
        ipcat0 = jnp.concatenate(diffuse(xt), axis=1)
        h0 = gru(ipcat0, h0, wi0, wg0s, bg0, wc0s, bc0)
        ipcat1 = jnp.concatenate(diffuse(h0), axis=1)
        h1 = gru(ipcat1, h1, wi1, wg1s, bg1, wc1s, bc1)

    lg = jnp.sum(jnp.maximum(h1, 0.0) * wfc_ref[...], axis=1,
                 keepdims=True) + bfc_ref[...]
    out_ref[...] = jnp.concatenate(
        [jnp.full((1, 1, _N), jnp.max(lg[i * _N:(i + 1) * _N]), jnp.float32)
         for i in range(_SPB)], axis=0)


def _split_w(w, din):
    wr = w.reshape(_NUM_MAT, din + _H, -1)
    w_in = wr[:, :din, :].reshape(_NUM_MAT * din, -1)
    w_st = wr[:, din:, :].reshape(_NUM_MAT * _H, -1)
    return w_in, w_st


@jax.jit
def kernel(x, W_gate0, b_gate0, W_cand0, b_cand0, W_gate1, b_gate1,
           W_cand1, b_cand1, W_fc, b_fc):
    wg0i, wg0s = _split_w(W_gate0, _D)
    wc0i, wc0s = _split_w(W_cand0, _D)
    wg1i, wg1s = _split_w(W_gate1, _H)
    wc1i, wc1s = _split_w(W_cand1, _H)
    wi0 = jnp.concatenate([wg0i, wc0i], axis=1)
    wi1 = jnp.concatenate([wg1i, wc1i], axis=1)

    const = lambda b: (0, 0)
    wspec = lambda a: pl.BlockSpec(a.shape, const)
    operands = (x, wi0, wg0s, b_gate0.reshape(1, -1),
                wc0s, b_cand0.reshape(1, -1),
                wi1, wg1s, b_gate1.reshape(1, -1),
                wc1s, b_cand1.reshape(1, -1),
                W_fc.reshape(1, _H), b_fc.reshape(1, 1))
    in_specs = [pl.BlockSpec((_SPB, _T, _N, _D), lambda b: (b, 0, 0, 0))]
    in_specs += [wspec(a) for a in operands[1:]]

    out = pl.pallas_call(
        _body,
        grid=(_B // _SPB,),
        in_specs=in_specs,
        out_specs=pl.BlockSpec((_SPB, 1, _N), lambda b: (b, 0, 0)),
        out_shape=jax.ShapeDtypeStruct((_B, 1, _N), jnp.float32),
        compiler_params=pltpu.CompilerParams(
            dimension_semantics=("parallel",)),
    )(*operands)
    return out[:, 0, 0]


# speedup vs baseline: 4.6040x; 1.2568x over previous
import jax
import jax.numpy as jnp
from jax.experimental import pallas as pl
from jax.experimental.pallas import tpu as pltpu

_B, _T, _N, _D, _H = 16, 12, 128, 64, 128
_NUM_MAT = 5
_SPB = 16


def _mm(a, b):
    return jax.lax.dot_general(a, b, (((1,), (0,)), ((), ())),
                               preferred_element_type=jnp.float32)


def _mmt(a, b):
    return jax.lax.dot_general(a, b, (((1,), (1,)), ((), ())),
                               preferred_element_type=jnp.float32)


def _body(x_ref, wi0_ref, wg0s_ref, bg0_ref, wc0s_ref, bc0_ref,
          wi1_ref, wg1s_ref, bg1_ref, wc1s_ref, bc1_ref,
          wfc_ref, bfc_ref, out_ref):
    row = jax.lax.broadcasted_iota(jnp.int32, (_N, _N), 0)
    col = jax.lax.broadcasted_iota(jnp.int32, (_N, _N), 1)
    eye = jnp.where(row == col, 1.0, 0.0).astype(jnp.float32)
    sup = []
    for i in range(_SPB):
        xc = x_ref[i]
        mu = jnp.sum(jnp.sum(xc, axis=0), axis=1, keepdims=True) \
            * (1.0 / (_T * _D))
        xm = xc - mu[None]
        cov = _mmt(xm[0], xm[0])
        for t in range(1, _T):
            cov = cov + _mmt(xm[t], xm[t])
        sq = jnp.sum(jnp.sum(xm * xm, axis=0), axis=1, keepdims=True)
        var = jnp.sqrt(jnp.maximum(sq, 1e-12))
        adj = jnp.abs(cov) / (var * jnp.transpose(var))
        s0 = adj / jnp.sum(adj, axis=1, keepdims=True)
        adjt = jnp.transpose(adj)
        s1 = adjt / jnp.transpose(jnp.sum(adj, axis=0, keepdims=True))
        sup.append(jnp.concatenate(
            [s0, _mm(s0, s0) * 2.0 - eye, s1, _mm(s1, s1) * 2.0 - eye],
            axis=0))

    def diffuse(v):
        prods = [_mm(sup[i], v[i * _N:(i + 1) * _N]) for i in range(_SPB)]
        return (v,) + tuple(
            jnp.concatenate([p[k * _N:(k + 1) * _N] for p in prods], axis=0)
            for k in range(4))

    wi0 = wi0_ref[...]
    wg0s = wg0s_ref[...]
    bg0 = bg0_ref[...]
    wc0s = wc0s_ref[...]
    bc0 = bc0_ref[...]
    wi1 = wi1_ref[...]
    wg1s = wg1s_ref[...]
    bg1 = bg1_ref[...]
    wc1s = wc1s_ref[...]
    bc1 = bc1_ref[...]

    def gru(ipcat, h, wi, wgs, bg, wcs, bc):
        ipg = _mm(ipcat, wi)
        stcat = jnp.concatenate(diffuse(h), axis=1)
        g = jax.nn.sigmoid(ipg[:, :2 * _H] + _mm(stcat, wgs) + bg)
        r = g[:, :_H]
        u = g[:, _H:]
        rh = r * h
        rcat = jnp.concatenate(diffuse(rh), axis=1)
        c = jnp.tanh(ipg[:, 2 * _H:] + _mm(rcat, wcs) + bc)
        return u * h + (1.0 - u) * c

    h0 = jnp.zeros((_SPB * _N, _H), jnp.float32)
    h1 = jnp.zeros((_SPB * _N, _H), jnp.float32)
    for t in range(_T):
        xt = jnp.concatenate([x_ref[i, t] for i in range(_SPB)], axis=0)
        ipcat0 = jnp.concatenate(diffuse(xt), axis=1)
        h0 = gru(ipcat0, h0, wi0, wg0s, bg0, wc0s, bc0)
        ipcat1 = jnp.concatenate(diffuse(h0), axis=1)
        h1 = gru(ipcat1, h1, wi1, wg1s, bg1, wc1s, bc1)

    lg = jnp.sum(jnp.maximum(h1, 0.0) * wfc_ref[...], axis=1,
                 keepdims=True) + bfc_ref[...]
    out_ref[...] = jnp.concatenate(
        [jnp.full((1, 1, _N), jnp.max(lg[i * _N:(i + 1) * _N]), jnp.float32)
         for i in range(_SPB)], axis=0)


def _split_w(w, din):
    wr = w.reshape(_NUM_MAT, din + _H, -1)
    w_in = wr[:, :din, :].reshape(_NUM_MAT * din, -1)
    w_st = wr[:, din:, :].reshape(_NUM_MAT * _H, -1)
    return w_in, w_st


@jax.jit
def kernel(x, W_gate0, b_gate0, W_cand0, b_cand0, W_gate1, b_gate1,
           W_cand1, b_cand1, W_fc, b_fc):
    wg0i, wg0s = _split_w(W_gate0, _D)
    wc0i, wc0s = _split_w(W_cand0, _D)
    wg1i, wg1s = _split_w(W_gate1, _H)
    wc1i, wc1s = _split_w(W_cand1, _H)
    wi0 = jnp.concatenate([wg0i, wc0i], axis=1)
    wi1 = jnp.concatenate([wg1i, wc1i], axis=1)

    const = lambda b: (0, 0)
    wspec = lambda a: pl.BlockSpec(a.shape, const)
    operands = (x, wi0, wg0s, b_gate0.reshape(1, -1),
                wc0s, b_cand0.reshape(1, -1),
                wi1, wg1s, b_gate1.reshape(1, -1),
                wc1s, b_cand1.reshape(1, -1),
                W_fc.reshape(1, _H), b_fc.reshape(1, 1))
    in_specs = [pl.BlockSpec((_SPB, _T, _N, _D), lambda b: (b, 0, 0, 0))]
    in_specs += [wspec(a) for a in operands[1:]]

    out = pl.pallas_call(
        _body,
        grid=(_B // _SPB,),
        in_specs=in_specs,
        out_specs=pl.BlockSpec((_SPB, 1, _N), lambda b: (b, 0, 0)),
        out_shape=jax.ShapeDtypeStruct((_B, 1, _N), jnp.float32),
        compiler_params=pltpu.CompilerParams(
            dimension_semantics=("parallel",)),
    )(*operands)
    return out[:, 0, 0]


# speedup vs baseline: 4.9408x; 1.0732x over previous
import jax
import jax.numpy as jnp
from jax.experimental import pallas as pl
from jax.experimental.pallas import tpu as pltpu

_B, _T, _N, _D, _H = 16, 12, 128, 64, 128
_NUM_MAT = 5
_SPB = 16


def _mm(a, b):
    return jax.lax.dot_general(a, b, (((1,), (0,)), ((), ())),
                               preferred_element_type=jnp.float32)


def _mmt(a, b):
    return jax.lax.dot_general(a, b, (((1,), (1,)), ((), ())),
                               preferred_element_type=jnp.float32)


def _body(x_ref, wi0_ref, wg0s_ref, bg0_ref, wc0s_ref, bc0_ref,
          wi1_ref, wg1s_ref, bg1_ref, wc1s_ref, bc1_ref,
          wfc_ref, bfc_ref, out_ref):
    row = jax.lax.broadcasted_iota(jnp.int32, (_N, _N), 0)
    col = jax.lax.broadcasted_iota(jnp.int32, (_N, _N), 1)
    eye = jnp.where(row == col, 1.0, 0.0).astype(jnp.float32)
    sup = []
    xs = []
    for i in range(_SPB):
        xb = x_ref[i]
        xs.append(xb)
        mu = jnp.sum(xb, axis=1, keepdims=True) * (1.0 / (_T * _D))
        xm = xb - mu
        cov = _mmt(xm, xm)
        sq = jnp.sum(xm * xm, axis=1, keepdims=True)
        var = jnp.sqrt(jnp.maximum(sq, 1e-12))
        adj = jnp.abs(cov) / (var * jnp.transpose(var))
        s0 = adj / jnp.sum(adj, axis=1, keepdims=True)
        adjt = jnp.transpose(adj)
        s1 = adjt / jnp.transpose(jnp.sum(adj, axis=0, keepdims=True))
        sup.append(jnp.concatenate(
            [s0, _mm(s0, s0) * 2.0 - eye, s1, _mm(s1, s1) * 2.0 - eye],
            axis=0))

    def diffuse(v):
        prods = [_mm(sup[i], v[i * _N:(i + 1) * _N]) for i in range(_SPB)]
        return (v,) + tuple(
            jnp.concatenate([p[k * _N:(k + 1) * _N] for p in prods], axis=0)
            for k in range(4))

    wi0 = wi0_ref[...]
    wg0s = wg0s_ref[...]
    bg0 = bg0_ref[...]
    wc0s = wc0s_ref[...]
    bc0 = bc0_ref[...]
    wi1 = wi1_ref[...]
    wg1s = wg1s_ref[...]
    bg1 = bg1_ref[...]
    wc1s = wc1s_ref[...]
    bc1 = bc1_ref[...]

    def gru(ipcat, h, wi, wgs, bg, wcs, bc):
        ipg = _mm(ipcat, wi)
        stcat = jnp.concatenate(diffuse(h), axis=1)
        g = jax.nn.sigmoid(ipg[:, :2 * _H] + _mm(stcat, wgs) + bg)
        r = g[:, :_H]
        u = g[:, _H:]
        rh = r * h
        rcat = jnp.concatenate(diffuse(rh), axis=1)
        c = jnp.tanh(ipg[:, 2 * _H:] + _mm(rcat, wcs) + bc)
        return u * h + (1.0 - u) * c

    h0 = jnp.zeros((_SPB * _N, _H), jnp.float32)
    h1 = jnp.zeros((_SPB * _N, _H), jnp.float32)
    for t in range(_T):
        sl = slice(t * _D, (t + 1) * _D)
        xt = jnp.concatenate([xs[i][:, sl] for i in range(_SPB)], axis=0)
        ipcat0 = jnp.concatenate(diffuse(xt), axis=1)
        h0 = gru(ipcat0, h0, wi0, wg0s, bg0, wc0s, bc0)
        ipcat1 = jnp.concatenate(diffuse(h0), axis=1)
        h1 = gru(ipcat1, h1, wi1, wg1s, bg1, wc1s, bc1)

    lg = jnp.sum(jnp.maximum(h1, 0.0) * wfc_ref[...], axis=1,
                 keepdims=True) + bfc_ref[...]
    out_ref[...] = jnp.concatenate(
        [jnp.full((1, 1, _N), jnp.max(lg[i * _N:(i + 1) * _N]), jnp.float32)
         for i in range(_SPB)], axis=0)


def _split_w(w, din):
    wr = w.reshape(_NUM_MAT, din + _H, -1)
    w_in = wr[:, :din, :].reshape(_NUM_MAT * din, -1)
    w_st = wr[:, din:, :].reshape(_NUM_MAT * _H, -1)
    return w_in, w_st


@jax.jit
def kernel(x, W_gate0, b_gate0, W_cand0, b_cand0, W_gate1, b_gate1,
           W_cand1, b_cand1, W_fc, b_fc):
    wg0i, wg0s = _split_w(W_gate0, _D)
    wc0i, wc0s = _split_w(W_cand0, _D)
    wg1i, wg1s = _split_w(W_gate1, _H)
    wc1i, wc1s = _split_w(W_cand1, _H)
    wi0 = jnp.concatenate([wg0i, wc0i], axis=1)
    wi1 = jnp.concatenate([wg1i, wc1i], axis=1)

    const = lambda b: (0, 0)
    wspec = lambda a: pl.BlockSpec(a.shape, const)
    xp = jnp.transpose(x, (0, 2, 1, 3)).reshape(_B, _N, _T * _D)
    operands = (xp, wi0, wg0s, b_gate0.reshape(1, -1),
                wc0s, b_cand0.reshape(1, -1),
                wi1, wg1s, b_gate1.reshape(1, -1),
                wc1s, b_cand1.reshape(1, -1),
                W_fc.reshape(1, _H), b_fc.reshape(1, 1))
    in_specs = [pl.BlockSpec((_SPB, _N, _T * _D), lambda b: (b, 0, 0))]
    in_specs += [wspec(a) for a in operands[1:]]

    out = pl.pallas_call(
        _body,
        grid=(_B // _SPB,),
        in_specs=in_specs,
        out_specs=pl.BlockSpec((_SPB, 1, _N), lambda b: (b, 0, 0)),
        out_shape=jax.ShapeDtypeStruct((_B, 1, _N), jnp.float32),
        compiler_params=pltpu.CompilerParams(
            dimension_semantics=("parallel",)),
    )(*operands)
    return out[:, 0, 0]


# speedup vs baseline: 5.1818x; 1.0488x over previous
import jax
import jax.numpy as jnp
from jax.experimental import pallas as pl
from jax.experimental.pallas import tpu as pltpu

_B, _T, _N, _D, _H = 16, 12, 128, 64, 128
_NUM_MAT = 5
_SPB = 16


def _mm(a, b):
    return jax.lax.dot_general(a, b, (((1,), (0,)), ((), ())),
                               preferred_element_type=jnp.float32)


def _mmt(a, b):
    return jax.lax.dot_general(a, b, (((1,), (1,)), ((), ())),
                               preferred_element_type=jnp.float32)


def _body(x_ref, wi0_ref, wg0s_ref, bg0_ref, wc0s_ref, bc0_ref,
          wi1_ref, wg1s_ref, bg1_ref, wc1s_ref, bc1_ref,
          wfc_ref, bfc_ref, out_ref):
    row = jax.lax.broadcasted_iota(jnp.int32, (_N, _N), 0)
    col = jax.lax.broadcasted_iota(jnp.int32, (_N, _N), 1)
    eye = jnp.where(row == col, 1.0, 0.0).astype(jnp.float32)
    sup = []
    xs = []
    for i in range(_SPB):
        xb = x_ref[i]
        xs.append(xb)
        mu = jnp.sum(xb, axis=1, keepdims=True) * (1.0 / (_T * _D))
        xm = xb - mu
        cov = _mmt(xm, xm)
        sq = jnp.sum(xm * xm, axis=1, keepdims=True)
        var = jnp.sqrt(jnp.maximum(sq, 1e-12))
        adj = jnp.abs(cov) / (var * jnp.transpose(var))
        s0 = adj / jnp.sum(adj, axis=1, keepdims=True)
        adjt = jnp.transpose(adj)
        s1 = adjt / jnp.transpose(jnp.sum(adj, axis=0, keepdims=True))
        sup.append(jnp.concatenate(
            [s0, _mm(s0, s0) * 2.0 - eye, s1, _mm(s1, s1) * 2.0 - eye],
            axis=0))

    def diffuse(v):
        prods = [_mm(sup[i], v[i * _N:(i + 1) * _N]) for i in range(_SPB)]
        return (v,) + tuple(
            jnp.concatenate([p[k * _N:(k + 1) * _N] for p in prods], axis=0)
            for k in range(4))

    wi0 = wi0_ref[...]
    wg0s = wg0s_ref[...]
    bg0 = bg0_ref[...]
    wc0s = wc0s_ref[...]
    bc0 = bc0_ref[...]
    wi1 = wi1_ref[...]
    wg1s = wg1s_ref[...]
    bg1 = bg1_ref[...]
    wc1s = wc1s_ref[...]
    bc1 = bc1_ref[...]

    def gru(ipcat, h, wi, wgs, bg, wcs, bc):
        ipg = _mm(ipcat, wi)
        stcat = jnp.concatenate(diffuse(h), axis=1)
        g = jax.nn.sigmoid(ipg[:, :2 * _H] + _mm(stcat, wgs) + bg)
        r = g[:, :_H]
        u = g[:, _H:]
        rh = r * h
        rcat = jnp.concatenate(diffuse(rh), axis=1)
        c = jnp.tanh(ipg[:, 2 * _H:] + _mm(rcat, wcs) + bc)
        return u * h + (1.0 - u) * c

    h0 = jnp.zeros((_SPB * _N, _H), jnp.float32)
    h1 = jnp.zeros((_SPB * _N, _H), jnp.float32)
    pxc = None
    for t in range(_T):
        if t % 4 == 0:
            csl = slice(t * _D, (t + 4) * _D)
            pxc = [_mm(sup[i], xs[i][:, csl]) for i in range(_SPB)]
        sl = slice(t * _D, (t + 1) * _D)
        sl4 = slice((t % 4) * _D, (t % 4 + 1) * _D)
        ipcat0 = jnp.concatenate(
            [jnp.concatenate([xs[i][:, sl] for i in range(_SPB)], axis=0)] +
            [jnp.concatenate([pxc[i][k * _N:(k + 1) * _N, sl4]
                              for i in range(_SPB)], axis=0)
             for k in range(4)], axis=1)
        h0 = gru(ipcat0, h0, wi0, wg0s, bg0, wc0s, bc0)
        ipcat1 = jnp.concatenate(diffuse(h0), axis=1)
        h1 = gru(ipcat1, h1, wi1, wg1s, bg1, wc1s, bc1)

    lg = jnp.sum(jnp.maximum(h1, 0.0) * wfc_ref[...], axis=1,
                 keepdims=True) + bfc_ref[...]
    out_ref[...] = jnp.concatenate(
        [jnp.full((1, 1, _N), jnp.max(lg[i * _N:(i + 1) * _N]), jnp.float32)
         for i in range(_SPB)], axis=0)


def _split_w(w, din):
    wr = w.reshape(_NUM_MAT, din + _H, -1)
    w_in = wr[:, :din, :].reshape(_NUM_MAT * din, -1)
    w_st = wr[:, din:, :].reshape(_NUM_MAT * _H, -1)
    return w_in, w_st


@jax.jit
def kernel(x, W_gate0, b_gate0, W_cand0, b_cand0, W_gate1, b_gate1,
           W_cand1, b_cand1, W_fc, b_fc):
    wg0i, wg0s = _split_w(W_gate0, _D)
    wc0i, wc0s = _split_w(W_cand0, _D)
    wg1i, wg1s = _split_w(W_gate1, _H)
    wc1i, wc1s = _split_w(W_cand1, _H)
    wi0 = jnp.concatenate([wg0i, wc0i], axis=1)
    wi1 = jnp.concatenate([wg1i, wc1i], axis=1)

    const = lambda b: (0, 0)
    wspec = lambda a: pl.BlockSpec(a.shape, const)
    xp = jnp.transpose(x, (0, 2, 1, 3)).reshape(_B, _N, _T * _D)
    operands = (xp, wi0, wg0s, b_gate0.reshape(1, -1),
                wc0s, b_cand0.reshape(1, -1),
                wi1, wg1s, b_gate1.reshape(1, -1),
                wc1s, b_cand1.reshape(1, -1),
                W_fc.reshape(1, _H), b_fc.reshape(1, 1))
    in_specs = [pl.BlockSpec((_SPB, _N, _T * _D), lambda b: (b, 0, 0))]
    in_specs += [wspec(a) for a in operands[1:]]

    out = pl.pallas_call(
        _body,
        grid=(_B // _SPB,),
        in_specs=in_specs,
        out_specs=pl.BlockSpec((_SPB, 1, _N), lambda b: (b, 0, 0)),
        out_shape=jax.ShapeDtypeStruct((_B, 1, _N), jnp.float32),
        compiler_params=pltpu.CompilerParams(
            dimension_semantics=("parallel",)),
    )(*operands)
    return out[:, 0, 0]


# speedup vs baseline: 5.3944x; 1.0410x over previous
import jax
import jax.numpy as jnp
from jax.experimental import pallas as pl
from jax.experimental.pallas import tpu as pltpu

_B, _T, _N, _D, _H = 16, 12, 128, 64, 128
_NUM_MAT = 5
_SPB = 16


def _mm(a, b):
    return jax.lax.dot_general(a, b, (((1,), (0,)), ((), ())),
                               preferred_element_type=jnp.float32)


def _mmt(a, b):
    return jax.lax.dot_general(a, b, (((1,), (1,)), ((), ())),
                               preferred_element_type=jnp.float32)


def _body(x_ref, wi0_ref, wg0s_ref, bg0_ref, wc0s_ref, bc0_ref,
          wi1_ref, wg1s_ref, bg1_ref, wc1s_ref, bc1_ref,
          wfc_ref, bfc_ref, out_ref):
    row = jax.lax.broadcasted_iota(jnp.int32, (_N, _N), 0)
    col = jax.lax.broadcasted_iota(jnp.int32, (_N, _N), 1)
    eye = jnp.where(row == col, 1.0, 0.0).astype(jnp.float32)
    sup = []
    xs = []
    for i in range(_SPB):
        xb = x_ref[i]
        xs.append(xb)
        mu = jnp.sum(xb, axis=1, keepdims=True) * (1.0 / (_T * _D))
        xm = xb - mu
        cov = _mmt(xm, xm)
        sq = jnp.sum(xm * xm, axis=1, keepdims=True)
        var = jnp.sqrt(jnp.maximum(sq, 1e-12))
        adj = jnp.abs(cov) / (var * jnp.transpose(var))
        s0 = adj / jnp.sum(adj, axis=1, keepdims=True)
        adjt = jnp.transpose(adj)
        s1 = adjt / jnp.transpose(jnp.sum(adj, axis=0, keepdims=True))
        sup.append(jnp.concatenate(
            [s0, _mm(s0, s0) * 2.0 - eye, s1, _mm(s1, s1) * 2.0 - eye],
            axis=0))

    def diffuse(v):
        prods = [_mm(sup[i], v[i * _N:(i + 1) * _N]) for i in range(_SPB)]
        return (v,) + tuple(
            jnp.concatenate([p[k * _N:(k + 1) * _N] for p in prods], axis=0)
            for k in range(4))

    wi0 = wi0_ref[...]
    wg0s = wg0s_ref[...]
    bg0 = bg0_ref[...]
    wc0s = wc0s_ref[...]
    bc0 = bc0_ref[...]
    wi1 = wi1_ref[...]
    wg1s = wg1s_ref[...]
    bg1 = bg1_ref[...]
    wc1s = wc1s_ref[...]
    bc1 = bc1_ref[...]

    def diffuse_pair(a, b):
        prods = [_mm(sup[i],
                     jnp.concatenate([a[i * _N:(i + 1) * _N],
                                      b[i * _N:(i + 1) * _N]], axis=1))
                 for i in range(_SPB)]
        da = tuple(
            jnp.concatenate([p[k * _N:(k + 1) * _N, :_H] for p in prods],
                            axis=0) for k in range(4))
        db = tuple(
            jnp.concatenate([p[k * _N:(k + 1) * _N, _H:] for p in prods],
                            axis=0) for k in range(4))
        return da, db

    def gru(ipcat, h, hdiff, wi, wgs, bg, wcs, bc):
        ipg = _mm(ipcat, wi)
        stcat = jnp.concatenate((h,) + hdiff, axis=1)
        g = jax.nn.sigmoid(ipg[:, :2 * _H] + _mm(stcat, wgs) + bg)
        r = g[:, :_H]
        u = g[:, _H:]
        rh = r * h
        rcat = jnp.concatenate(diffuse(rh), axis=1)
        c = jnp.tanh(ipg[:, 2 * _H:] + _mm(rcat, wcs) + bc)
        return u * h + (1.0 - u) * c

    def ipcat0_at(t, pxc):
        sl = slice(t * _D, (t + 1) * _D)
        sl4 = slice((t % 4) * _D, (t % 4 + 1) * _D)
        return jnp.concatenate(
            [jnp.concatenate([xs[i][:, sl] for i in range(_SPB)], axis=0)] +
            [jnp.concatenate([pxc[i][k * _N:(k + 1) * _N, sl4]
                              for i in range(_SPB)], axis=0)
             for k in range(4)], axis=1)

    pxc = [_mm(sup[i], xs[i][:, 0:4 * _D]) for i in range(_SPB)]
    ipg = _mm(ipcat0_at(0, pxc), wi0)
    u = jax.nn.sigmoid(ipg[:, _H:2 * _H] + bg0[:, _H:])
    c = jnp.tanh(ipg[:, 2 * _H:] + bc0)
    h0 = (1.0 - u) * c
    h0diff = diffuse(h0)[1:]
    ipg1 = _mm(jnp.concatenate((h0,) + h0diff, axis=1), wi1)
    u1 = jax.nn.sigmoid(ipg1[:, _H:2 * _H] + bg1[:, _H:])
    c1 = jnp.tanh(ipg1[:, 2 * _H:] + bc1)
    h1 = (1.0 - u1) * c1

    for t in range(1, _T):
        if t % 4 == 0:
            csl = slice(t * _D, (t + 4) * _D)
            pxc = [_mm(sup[i], xs[i][:, csl]) for i in range(_SPB)]
        h0_new = gru(ipcat0_at(t, pxc), h0, h0diff,
                     wi0, wg0s, bg0, wc0s, bc0)
        h0diff, h1diff = diffuse_pair(h0_new, h1)
        ipcat1 = jnp.concatenate((h0_new,) + h0diff, axis=1)
        h1 = gru(ipcat1, h1, h1diff, wi1, wg1s, bg1, wc1s, bc1)
        h0 = h0_new

    lg = jnp.sum(jnp.maximum(h1, 0.0) * wfc_ref[...], axis=1,
                 keepdims=True) + bfc_ref[...]
    out_ref[...] = jnp.concatenate(
        [jnp.full((1, 1, _N), jnp.max(lg[i * _N:(i + 1) * _N]), jnp.float32)
         for i in range(_SPB)], axis=0)


def _split_w(w, din):
    wr = w.reshape(_NUM_MAT, din + _H, -1)
    w_in = wr[:, :din, :].reshape(_NUM_MAT * din, -1)
    w_st = wr[:, din:, :].reshape(_NUM_MAT * _H, -1)
    return w_in, w_st


@jax.jit
def kernel(x, W_gate0, b_gate0, W_cand0, b_cand0, W_gate1, b_gate1,
           W_cand1, b_cand1, W_fc, b_fc):
    wg0i, wg0s = _split_w(W_gate0, _D)
    wc0i, wc0s = _split_w(W_cand0, _D)
    wg1i, wg1s = _split_w(W_gate1, _H)
    wc1i, wc1s = _split_w(W_cand1, _H)
    wi0 = jnp.concatenate([wg0i, wc0i], axis=1)
    wi1 = jnp.concatenate([wg1i, wc1i], axis=1)

    const = lambda b: (0, 0)
    wspec = lambda a: pl.BlockSpec(a.shape, const)
    xp = jnp.transpose(x, (0, 2, 1, 3)).reshape(_B, _N, _T * _D)
    operands = (xp, wi0, wg0s, b_gate0.reshape(1, -1),
                wc0s, b_cand0.reshape(1, -1),
                wi1, wg1s, b_gate1.reshape(1, -1),
                wc1s, b_cand1.reshape(1, -1),
                W_fc.reshape(1, _H), b_fc.reshape(1, 1))
    in_specs = [pl.BlockSpec((_SPB, _N, _T * _D), lambda b: (b, 0, 0))]
    in_specs += [wspec(a) for a in operands[1:]]

    out = pl.pallas_call(
        _body,
        grid=(_B // _SPB,),
        in_specs=in_specs,
        out_specs=pl.BlockSpec((_SPB, 1, _N), lambda b: (b, 0, 0)),
        out_shape=jax.ShapeDtypeStruct((_B, 1, _N), jnp.float32),
        compiler_params=pltpu.CompilerParams(
            dimension_semantics=("parallel",)),
    )(*operands)
    return out[:, 0, 0]


# speedup vs baseline: 7.1671x; 1.3286x over previous
import jax
import jax.numpy as jnp
from jax.experimental import pallas as pl
from jax.experimental.pallas import tpu as pltpu

_B, _T, _N, _D, _H = 16, 12, 128, 64, 128
_NUM_MAT = 5
_M = _B * _N


def _mm(a, b):
    return jax.lax.dot_general(a, b, (((1,), (0,)), ((), ())),
                               preferred_element_type=jnp.float32)


def _mmr(a, b):
    return jax.lax.dot_general(a, b, (((0,), (0,)), ((), ())),
                               preferred_element_type=jnp.float32)


def _body(x_ref, wi0_ref, wg0s_ref, bg0_ref, wc0s_ref, bc0_ref,
          wi1_ref, wg1s_ref, bg1_ref, wc1s_ref, bc1_ref,
          wfc_ref, bfc_ref, out_ref):
    row = jax.lax.broadcasted_iota(jnp.int32, (_N, _N), 0)
    col = jax.lax.broadcasted_iota(jnp.int32, (_N, _N), 1)
    eye = jnp.where(row == col, 1.0, 0.0).astype(jnp.float32)
    sup = []
    xs = []
    for i in range(_B):
        xt_i = x_ref[i]
        xs.append(xt_i)
        mu = jnp.sum(xt_i, axis=0, keepdims=True) * (1.0 / (_T * _D))
        xm = xt_i - mu
        cov = _mmr(xm, xm)
        sq = jnp.sum(xm * xm, axis=0, keepdims=True)
        var = jnp.sqrt(jnp.maximum(sq, 1e-12))
        adj = jnp.abs(cov) / (jnp.transpose(var) * var)
        rs = jnp.sum(adj, axis=1, keepdims=True)
        cs = jnp.sum(adj, axis=0, keepdims=True)
        s0t = jnp.transpose(adj) / jnp.transpose(rs)
        s1t = adj / cs
        sup.append(jnp.concatenate(
            [s0t, _mm(s0t, s0t) * 2.0 - eye, s1t,
             _mm(s1t, s1t) * 2.0 - eye], axis=1))

    def diffuse(v):
        prods = [_mm(v[:, i * _N:(i + 1) * _N], sup[i]) for i in range(_B)]
        return tuple(
            jnp.concatenate([p[:, k * _N:(k + 1) * _N] for p in prods],
                            axis=1) for k in range(4))

    wi0 = wi0_ref[...]
    wg0s = wg0s_ref[...]
    bg0 = bg0_ref[...]
    wc0s = wc0s_ref[...]
    bc0 = bc0_ref[...]
    wi1 = wi1_ref[...]
    wg1s = wg1s_ref[...]
    bg1 = bg1_ref[...]
    wc1s = wc1s_ref[...]
    bc1 = bc1_ref[...]

    def gru(ipcat, h, hdiff, wi, wgs, bg, wcs, bc):
        ipg = _mm(wi, ipcat)
        stcat = jnp.concatenate((h,) + hdiff, axis=0)
        g = jax.nn.sigmoid(ipg[:2 * _H] + _mm(wgs, stcat) + bg)
        r = g[:_H]
        u = g[_H:]
        rh = r * h
        rcat = jnp.concatenate((rh,) + diffuse(rh), axis=0)
        c = jnp.tanh(ipg[2 * _H:] + _mm(wcs, rcat) + bc)
        return u * h + (1.0 - u) * c

    def ipcat0_at(t, pxc):
        sl = slice(t * _D, (t + 1) * _D)
        sl4 = slice((t % 4) * _D, (t % 4 + 1) * _D)
        return jnp.concatenate(
            [jnp.concatenate([xs[i][sl] for i in range(_B)], axis=1)] +
            [jnp.concatenate([pxc[i][sl4, k * _N:(k + 1) * _N]
                              for i in range(_B)], axis=1)
             for k in range(4)], axis=0)

    def xchunk(t):
        csl = slice(t * _D, (t + 4) * _D)
        return [_mm(xs[i][csl], sup[i]) for i in range(_B)]

    pxc = xchunk(0)
    ipg = _mm(wi0, ipcat0_at(0, pxc))
    u = jax.nn.sigmoid(ipg[_H:2 * _H] + bg0[_H:])
    c = jnp.tanh(ipg[2 * _H:] + bc0)
    h0 = (1.0 - u) * c
    h0diff = diffuse(h0)
    ipg1 = _mm(wi1, jnp.concatenate((h0,) + h0diff, axis=0))
    u1 = jax.nn.sigmoid(ipg1[_H:2 * _H] + bg1[_H:])
    c1 = jnp.tanh(ipg1[2 * _H:] + bc1)
    h1 = (1.0 - u1) * c1

    for t in range(1, _T):
        if t % 4 == 0:
            pxc = xchunk(t)
        h0_new = gru(ipcat0_at(t, pxc), h0, h0diff,
                     wi0, wg0s, bg0, wc0s, bc0)
        both = jnp.concatenate([h0_new, h1], axis=0)
        bdiff = diffuse(both)
        h0diff = tuple(d[:_H] for d in bdiff)
        h1diff = tuple(d[_H:] for d in bdiff)
        ipcat1 = jnp.concatenate((h0_new,) + h0diff, axis=0)
        h1 = gru(ipcat1, h1, h1diff, wi1, wg1s, bg1, wc1s, bc1)
        h0 = h0_new

    lg = jnp.sum(jnp.maximum(h1, 0.0) * wfc_ref[...], axis=0,
                 keepdims=True) + bfc_ref[...]
    out_ref[...] = jnp.concatenate(
        [jnp.full((1, 1, _N), jnp.max(lg[:, i * _N:(i + 1) * _N]),
                  jnp.float32) for i in range(_B)], axis=0)


def _split_w(w, din):
    wr = w.reshape(_NUM_MAT, din + _H, -1)
    w_in = wr[:, :din, :].reshape(_NUM_MAT * din, -1)
    w_st = wr[:, din:, :].reshape(_NUM_MAT * _H, -1)
    return w_in, w_st


@jax.jit
def kernel(x, W_gate0, b_gate0, W_cand0, b_cand0, W_gate1, b_gate1,
           W_cand1, b_cand1, W_fc, b_fc):
    wg0i, wg0s = _split_w(W_gate0, _D)
    wc0i, wc0s = _split_w(W_cand0, _D)
    wg1i, wg1s = _split_w(W_gate1, _H)
    wc1i, wc1s = _split_w(W_cand1, _H)
    wi0 = jnp.concatenate([wg0i, wc0i], axis=1).T
    wi1 = jnp.concatenate([wg1i, wc1i], axis=1).T

    xp = jnp.transpose(x, (0, 1, 3, 2)).reshape(_B, _T * _D, _N)

    const = lambda b: (0, 0)
    wspec = lambda a: pl.BlockSpec(a.shape, const)
    operands = (xp, wi0, wg0s.T, b_gate0.reshape(-1, 1),
                wc0s.T, b_cand0.reshape(-1, 1),
                wi1, wg1s.T, b_gate1.reshape(-1, 1),
                wc1s.T, b_cand1.reshape(-1, 1),
                W_fc, b_fc.reshape(1, 1))
    in_specs = [pl.BlockSpec((_B, _T * _D, _N), lambda b: (b, 0, 0))]
    in_specs += [wspec(a) for a in operands[1:]]

    out = pl.pallas_call(
        _body,
        grid=(1,),
        in_specs=in_specs,
        out_specs=pl.BlockSpec((_B, 1, _N), lambda b: (b, 0, 0)),
        out_shape=jax.ShapeDtypeStruct((_B, 1, _N), jnp.float32),
        compiler_params=pltpu.CompilerParams(
            dimension_semantics=("arbitrary",)),
    )(*operands)
    return out[:, 0, 0]


# speedup vs baseline: 7.4810x; 1.0438x over previous
import jax
import jax.numpy as jnp
from jax.experimental import pallas as pl
from jax.experimental.pallas import tpu as pltpu

_B, _T, _N, _D, _H = 16, 12, 128, 64, 128
_NUM_MAT = 5
_M = _B * _N


def _mm(a, b):
    return jax.lax.dot_general(a, b, (((1,), (0,)), ((), ())),
                               preferred_element_type=jnp.float32)


def _mmr(a, b):
    return jax.lax.dot_general(a, b, (((0,), (0,)), ((), ())),
                               preferred_element_type=jnp.float32)


def _body(x_ref, wi0_ref, wg0s_ref, bg0_ref, wc0s_ref, bc0_ref,
          wi1_ref, wg1s_ref, bg1_ref, wc1s_ref, bc1_ref,
          wfc_ref, bfc_ref, out_ref):
    row = jax.lax.broadcasted_iota(jnp.int32, (_N, _N), 0)
    col = jax.lax.broadcasted_iota(jnp.int32, (_N, _N), 1)
    eye = jnp.where(row == col, 1.0, 0.0).astype(jnp.float32)
    sup = []
    xs = []
    for i in range(_B):
        xt_i = x_ref[i]
        xs.append(xt_i)
        mu = jnp.sum(xt_i, axis=0, keepdims=True) * (1.0 / (_T * _D))
        xm = xt_i - mu
        cov = _mmr(xm, xm)
        sq = jnp.sum(xm * xm, axis=0, keepdims=True)
        var = jnp.sqrt(jnp.maximum(sq, 1e-12))
        adj = jnp.abs(cov) / (jnp.transpose(var) * var)
        rs = jnp.sum(adj, axis=1, keepdims=True)
        cs = jnp.sum(adj, axis=0, keepdims=True)
        s0t = jnp.transpose(adj) / jnp.transpose(rs)
        s1t = adj / cs
        sup.append(jnp.concatenate(
            [s0t, _mm(s0t, s0t) * 2.0 - eye, s1t,
             _mm(s1t, s1t) * 2.0 - eye], axis=1))

    def diffuse(v):
        prods = [_mm(v[:, i * _N:(i + 1) * _N], sup[i]) for i in range(_B)]
        return tuple(
            jnp.concatenate([p[:, k * _N:(k + 1) * _N] for p in prods],
                            axis=1) for k in range(4))

    wi0 = wi0_ref[...]
    wg0s = wg0s_ref[...]
    bg0 = bg0_ref[...]
    wc0s = wc0s_ref[...]
    bc0 = bc0_ref[...]
    wi1 = wi1_ref[...]
    wg1s = wg1s_ref[...]
    bg1 = bg1_ref[...]
    wc1s = wc1s_ref[...]
    bc1 = bc1_ref[...]

    def gru(ipcat, h, hdiff, wi, wgs, bg, wcs, bc):
        ipg = _mm(wi, jnp.concatenate((ipcat, h), axis=0))
        dcat = jnp.concatenate(hdiff, axis=0)
        g = jax.nn.sigmoid(ipg[:2 * _H] + _mm(wgs, dcat) + bg)
        r = g[:_H]
        u = g[_H:]
        rh = r * h
        rcat = jnp.concatenate((rh,) + diffuse(rh), axis=0)
        c = jnp.tanh(ipg[2 * _H:] + _mm(wcs, rcat) + bc)
        return u * h + (1.0 - u) * c

    def ipcat0_at(t, pxc):
        sl = slice(t * _D, (t + 1) * _D)
        sl4 = slice((t % 4) * _D, (t % 4 + 1) * _D)
        return jnp.concatenate(
            [jnp.concatenate([xs[i][sl] for i in range(_B)], axis=1)] +
            [jnp.concatenate([pxc[i][sl4, k * _N:(k + 1) * _N]
                              for i in range(_B)], axis=1)
             for k in range(4)], axis=0)

    def xchunk(t):
        csl = slice(t * _D, (t + 4) * _D)
        return [_mm(xs[i][csl], sup[i]) for i in range(_B)]

    pxc = xchunk(0)
    ipg = _mm(wi0[:, :_NUM_MAT * _D], ipcat0_at(0, pxc))
    u = jax.nn.sigmoid(ipg[_H:2 * _H] + bg0[_H:])
    c = jnp.tanh(ipg[2 * _H:] + bc0)
    h0 = (1.0 - u) * c
    h0diff = diffuse(h0)
    ipg1 = _mm(wi1[:, :_NUM_MAT * _H],
               jnp.concatenate((h0,) + h0diff, axis=0))
    u1 = jax.nn.sigmoid(ipg1[_H:2 * _H] + bg1[_H:])
    c1 = jnp.tanh(ipg1[2 * _H:] + bc1)
    h1 = (1.0 - u1) * c1

    for t in range(1, _T):
        if t % 4 == 0:
            pxc = xchunk(t)
        h0_new = gru(ipcat0_at(t, pxc), h0, h0diff,
                     wi0, wg0s, bg0, wc0s, bc0)
        both = jnp.concatenate([h0_new, h1], axis=0)
        bdiff = diffuse(both)
        h0diff = tuple(d[:_H] for d in bdiff)
        h1diff = tuple(d[_H:] for d in bdiff)
        ipcat1 = jnp.concatenate((h0_new,) + h0diff, axis=0)
        h1 = gru(ipcat1, h1, h1diff, wi1, wg1s, bg1, wc1s, bc1)
        h0 = h0_new

    lg = jnp.sum(jnp.maximum(h1, 0.0) * wfc_ref[...], axis=0,
                 keepdims=True) + bfc_ref[...]
    out_ref[...] = jnp.concatenate(
        [jnp.full((1, 1, _N), jnp.max(lg[:, i * _N:(i + 1) * _N]),
                  jnp.float32) for i in range(_B)], axis=0)


def _split_w(w, din):
    wr = w.reshape(_NUM_MAT, din + _H, -1)
    w_in = wr[:, :din, :].reshape(_NUM_MAT * din, -1)
    w_st = wr[:, din:, :].reshape(_NUM_MAT * _H, -1)
    return w_in, w_st


@jax.jit
def kernel(x, W_gate0, b_gate0, W_cand0, b_cand0, W_gate1, b_gate1,
           W_cand1, b_cand1, W_fc, b_fc):
    wg0i, wg0s = _split_w(W_gate0, _D)
    wc0i, wc0s = _split_w(W_cand0, _D)
    wg1i, wg1s = _split_w(W_gate1, _H)
    wc1i, wc1s = _split_w(W_cand1, _H)
    wi0 = jnp.concatenate([wg0i, wc0i], axis=1).T
    wi1 = jnp.concatenate([wg1i, wc1i], axis=1).T
    id0 = jnp.concatenate([wg0s[:_H], jnp.zeros((_H, _H), jnp.float32)],
                          axis=1).T
    id1 = jnp.concatenate([wg1s[:_H], jnp.zeros((_H, _H), jnp.float32)],
                          axis=1).T
    wi0 = jnp.concatenate([wi0, id0], axis=1)
    wi1 = jnp.concatenate([wi1, id1], axis=1)
    wg0s = wg0s[_H:]
    wg1s = wg1s[_H:]

    xp = jnp.transpose(x, (0, 1, 3, 2)).reshape(_B, _T * _D, _N)

    const = lambda b: (0, 0)
    wspec = lambda a: pl.BlockSpec(a.shape, const)
    operands = (xp, wi0, wg0s.T, b_gate0.reshape(-1, 1),
                wc0s.T, b_cand0.reshape(-1, 1),
                wi1, wg1s.T, b_gate1.reshape(-1, 1),
                wc1s.T, b_cand1.reshape(-1, 1),
                W_fc, b_fc.reshape(1, 1))
    in_specs = [pl.BlockSpec((_B, _T * _D, _N), lambda b: (b, 0, 0))]
    in_specs += [wspec(a) for a in operands[1:]]

    out = pl.pallas_call(
        _body,
        grid=(1,),
        in_specs=in_specs,
        out_specs=pl.BlockSpec((_B, 1, _N), lambda b: (b, 0, 0)),
        out_shape=jax.ShapeDtypeStruct((_B, 1, _N), jnp.float32),
        compiler_params=pltpu.CompilerParams(
            dimension_semantics=("arbitrary",)),
    )(*operands)
    return out[:, 0, 0]
